# baseline (speedup 1.0000x reference)
import numpy as np
import jax
import jax.numpy as jnp
from jax.experimental import pallas as pl
from jax.experimental.pallas import tpu as pltpu

_P = 1.0
_FREQ_MASK_PARAM = 27
_TIME_MASK_PARAM = 100
_FREQ_MASKS = 2
_TIME_MASKS = 2

_G = 4
_NSLOT = 2 * _G


def _mask_vectors(batch, n_freq, n_time):
    rng = np.random.default_rng(0)
    if rng.random() > _P:
        return None
    rowm = np.zeros((batch, n_freq), np.float32)
    colm = np.zeros((batch, n_time), np.float32)
    for idx in range(batch):
        for _ in range(_FREQ_MASKS):
            max_w = min(_FREQ_MASK_PARAM, n_freq)
            w = int(rng.integers(0, max_w + 1))
            if w > 0:
                s = int(rng.integers(0, n_freq - w + 1))
                rowm[idx, s:s + w] = 1.0
        for _ in range(_TIME_MASKS):
            max_w = min(_TIME_MASK_PARAM, n_time)
            w = int(rng.integers(0, max_w + 1))
            if w > 0:
                s = int(rng.integers(0, n_time - w + 1))
                colm[idx, s:s + w] = 1.0
    return rowm, colm


def _make_body(batch, n_freq, n_time):
    n_elem = float(n_freq * n_time)
    n_groups = batch // _G

    def _group_loads(x_hbm, buf, sems, g, half):
        for u in range(_G):
            pltpu.async_copy(
                x_hbm.at[g * _G + u, 0],
                buf.at[half * _G + u],
                sems.at[half],
                priority=u % 2,
            )

    def _group_stores(o_hbm, buf, sems, g, half):
        for u in range(_G):
            pltpu.async_copy(
                buf.at[half * _G + u],
                o_hbm.at[g * _G + u, 0],
                sems.at[half],
                priority=u % 2,
            )

    def _group_wait_loads(x_hbm, buf, sems, g, half):
        pltpu.make_async_copy(
            x_hbm.at[pl.ds(g * _G, _G), 0],
            buf.at[pl.ds(half * _G, _G)],
            sems.at[half],
        ).wait()

    def _group_wait_stores(o_hbm, buf, sems, g, half):
        pltpu.make_async_copy(
            buf.at[pl.ds(half * _G, _G)],
            o_hbm.at[pl.ds(g * _G, _G), 0],
            sems.at[half],
        ).wait()

    def _body(x_hbm, rowm_ref, colm_ref, o_hbm, buf, load_sems, store_sems):
        s = pl.program_id(0)
        half = jax.lax.rem(s, 2)

        @pl.when(s < n_groups)
        def _issue_loads():
            @pl.when(s >= 2)
            def _():
                _group_wait_stores(o_hbm, buf, store_sems, s - 2, half)

            _group_loads(x_hbm, buf, load_sems, s, half)

        g = s - 1
        ghalf = jax.lax.rem(g, 2)

        @pl.when(g >= 0)
        def _process():
            _group_wait_loads(x_hbm, buf, load_sems, g, ghalf)
            for u in range(_G):
                slot = ghalf * _G + u
                j = g * _G + u
                xb = buf[slot]
                fill = jnp.sum(xb) * (1.0 / n_elem)
                rm = rowm_ref[j, 0, :]
                cm = colm_ref[j, 0, :]
                m = jnp.maximum(rm[:, None], cm[None, :]) > 0.0
                buf[slot] = jnp.where(m, fill, xb)
            _group_stores(o_hbm, buf, store_sems, g, ghalf)

        @pl.when(s == n_groups)
        def _drain():
            _group_wait_stores(o_hbm, buf, store_sems, n_groups - 2,
                               (n_groups - 2) % 2)
            _group_wait_stores(o_hbm, buf, store_sems, n_groups - 1,
                               (n_groups - 1) % 2)

    return _body


def kernel(x):
    batch, ch, n_freq, n_time = x.shape
    masks = _mask_vectors(batch, n_freq, n_time)
    if masks is None:
        return x
    rowm_np, colm_np = masks
    rowm = jnp.asarray(rowm_np).reshape(batch, 1, n_freq)
    colm = jnp.asarray(colm_np).reshape(batch, 1, n_time)
    n_groups = batch // _G

    out = pl.pallas_call(
        _make_body(batch, n_freq, n_time),
        grid=(n_groups + 1,),
        in_specs=[
            pl.BlockSpec(memory_space=pltpu.MemorySpace.HBM),
            pl.BlockSpec(memory_space=pltpu.MemorySpace.VMEM),
            pl.BlockSpec(memory_space=pltpu.MemorySpace.VMEM),
        ],
        out_specs=pl.BlockSpec(memory_space=pltpu.MemorySpace.HBM),
        out_shape=jax.ShapeDtypeStruct(x.shape, x.dtype),
        scratch_shapes=[
            pltpu.VMEM((_NSLOT, n_freq, n_time), x.dtype),
            pltpu.SemaphoreType.DMA((2,)),
            pltpu.SemaphoreType.DMA((2,)),
        ],
    )(x, rowm, colm)
    return out


# speedup vs baseline: 2.3344x; 2.3344x over previous
import numpy as np
import jax
from jax import lax
import jax.numpy as jnp
from jax.experimental import pallas as pl
from jax.experimental.pallas import tpu as pltpu

_P = 1.0
_FREQ_MASK_PARAM = 27
_TIME_MASK_PARAM = 100
_FREQ_MASKS = 2
_TIME_MASKS = 2

_G = 4
_NSLOT = 2 * _G


def _mask_bounds(batch, n_freq, n_time):
    rng = np.random.default_rng(0)
    if rng.random() > _P:
        return None
    bounds = np.zeros((batch, 8), np.int32)
    for idx in range(batch):
        for k in range(_FREQ_MASKS):
            max_w = min(_FREQ_MASK_PARAM, n_freq)
            w = int(rng.integers(0, max_w + 1))
            s = int(rng.integers(0, n_freq - w + 1)) if w > 0 else 0
            bounds[idx, 2 * k] = s
            bounds[idx, 2 * k + 1] = s + w
        for k in range(_TIME_MASKS):
            max_w = min(_TIME_MASK_PARAM, n_time)
            w = int(rng.integers(0, max_w + 1))
            s = int(rng.integers(0, n_time - w + 1)) if w > 0 else 0
            bounds[idx, 4 + 2 * k] = s
            bounds[idx, 4 + 2 * k + 1] = s + w
    return bounds


def _make_body(batch, n_freq, n_time):
    n_elem = float(n_freq * n_time)
    n_groups = batch // _G

    def _group_loads(x_hbm, buf, sems, g, half):
        for u in range(_G):
            pltpu.async_copy(
                x_hbm.at[g * _G + u, 0],
                buf.at[half * _G + u],
                sems.at[half],
                priority=u % 2,
            )

    def _group_stores(o_hbm, buf, sems, g, half):
        for u in range(_G):
            pltpu.async_copy(
                buf.at[half * _G + u],
                o_hbm.at[g * _G + u, 0],
                sems.at[half],
                priority=u % 2,
            )

    def _group_wait_loads(x_hbm, buf, sems, g, half):
        pltpu.make_async_copy(
            x_hbm.at[pl.ds(g * _G, _G), 0],
            buf.at[pl.ds(half * _G, _G)],
            sems.at[half],
        ).wait()

    def _group_wait_stores(o_hbm, buf, sems, g, half):
        pltpu.make_async_copy(
            buf.at[pl.ds(half * _G, _G)],
            o_hbm.at[pl.ds(g * _G, _G), 0],
            sems.at[half],
        ).wait()

    def _body(bounds_ref, x_hbm, o_hbm, buf, load_sems, store_sems):
        s = pl.program_id(0)
        half = jax.lax.rem(s, 2)

        @pl.when(s < n_groups)
        def _issue_loads():
            @pl.when(s >= 2)
            def _():
                _group_wait_stores(o_hbm, buf, store_sems, s - 2, half)

            _group_loads(x_hbm, buf, load_sems, s, half)

        g = s - 1
        ghalf = jax.lax.rem(g, 2)

        @pl.when(g >= 0)
        def _process():
            _group_wait_loads(x_hbm, buf, load_sems, g, ghalf)
            ti = lax.broadcasted_iota(jnp.int32, (n_time, n_freq), 0)
            fi = lax.broadcasted_iota(jnp.int32, (n_time, n_freq), 1)
            for u in range(_G):
                slot = ghalf * _G + u
                j = g * _G + u
                xb = buf[slot]
                fill = jnp.sum(xb) * (1.0 / n_elem)
                fs0 = bounds_ref[j, 0]
                fe0 = bounds_ref[j, 1]
                fs1 = bounds_ref[j, 2]
                fe1 = bounds_ref[j, 3]
                ts0 = bounds_ref[j, 4]
                te0 = bounds_ref[j, 5]
                ts1 = bounds_ref[j, 6]
                te1 = bounds_ref[j, 7]
                mf = ((fi >= fs0) & (fi < fe0)) | ((fi >= fs1) & (fi < fe1))
                mt = ((ti >= ts0) & (ti < te0)) | ((ti >= ts1) & (ti < te1))
                buf[slot] = jnp.where(mf | mt, fill, xb)
            _group_stores(o_hbm, buf, store_sems, g, ghalf)

        @pl.when(s == n_groups)
        def _drain():
            _group_wait_stores(o_hbm, buf, store_sems, n_groups - 2,
                               (n_groups - 2) % 2)
            _group_wait_stores(o_hbm, buf, store_sems, n_groups - 1,
                               (n_groups - 1) % 2)

    return _body


def kernel(x):
    batch, ch, n_freq, n_time = x.shape
    bounds_np = _mask_bounds(batch, n_freq, n_time)
    if bounds_np is None:
        return x
    bounds = jnp.asarray(bounds_np)
    n_groups = batch // _G

    xt = jnp.transpose(x, (0, 1, 3, 2))

    out_t = pl.pallas_call(
        _make_body(batch, n_freq, n_time),
        grid=(n_groups + 1,),
        in_specs=[
            pl.BlockSpec(memory_space=pltpu.MemorySpace.SMEM),
            pl.BlockSpec(memory_space=pltpu.MemorySpace.HBM),
        ],
        out_specs=pl.BlockSpec(memory_space=pltpu.MemorySpace.HBM),
        out_shape=jax.ShapeDtypeStruct(xt.shape, x.dtype),
        scratch_shapes=[
            pltpu.VMEM((_NSLOT, n_time, n_freq), x.dtype),
            pltpu.SemaphoreType.DMA((2,)),
            pltpu.SemaphoreType.DMA((2,)),
        ],
    )(bounds, xt)
    return jnp.transpose(out_t, (0, 1, 3, 2))
